# Initial kernel scaffold; baseline (speedup 1.0000x reference)
import jax
import jax.numpy as jnp
from jax.experimental import pallas as pl


def kernel(x, edge_index, W, b):
    raise NotImplementedError("write your pallas kernel here")


# speedup vs baseline: 10.1870x; 10.1870x over previous
import functools

import jax
import jax.numpy as jnp
from jax import lax
from jax.experimental import pallas as pl
from jax.experimental.pallas import tpu as pltpu
from jax.experimental.pallas import tpu_sc as plsc

N = 10000
E = 320000
D = 128
DH = D // 2

NC = 2
NS = 16
EPT = E // NS
CHUNK = 128
NFULL = EPT // CHUNK
TAIL = EPT - NFULL * CHUNK
DEG_SPLIT = NFULL // 2
ROWS_PER_TILE = N // NS


def _sc_body(xs_hbm, src_hbm, dst_hbm, agg_hbm, deg_hbm,
             src_idx, dst_idx, rows, tsrc_idx, tdst_idx, trows,
             ones_mat, zrow, zdeg, agg_sh, deg_sh, sem):
    cid = lax.axis_index("c")
    sid = lax.axis_index("s")

    zero16 = jnp.zeros((16,), jnp.float32)
    ones16 = jnp.full((16,), 1.0, jnp.float32)

    def init_row(i, _):
        ones_mat[i, :] = ones16
        zdeg[i, :] = zero16
        for j in range(DH // 16):
            zrow[i, pl.ds(j * 16, 16)] = zero16
        return _
    lax.fori_loop(0, CHUNK, init_row, None)

    base_row = sid * ROWS_PER_TILE
    off = 0
    for sz in (128, 128, 128, 128, 113):
        pltpu.sync_copy(zrow.at[pl.ds(0, sz)],
                        agg_sh.at[pl.ds(base_row + off, sz)])
        pltpu.sync_copy(zdeg.at[pl.ds(0, sz)],
                        deg_sh.at[pl.ds(base_row + off, sz)])
        off += sz
    plsc.subcore_barrier()

    ebase = sid * EPT
    my_half = xs_hbm.at[cid]

    def chunk_body(ci, _):
        eoff = ebase + ci * CHUNK
        pltpu.sync_copy(src_hbm.at[pl.ds(eoff, CHUNK)], src_idx)
        pltpu.sync_copy(dst_hbm.at[pl.ds(eoff, CHUNK)], dst_idx)
        pltpu.async_copy(my_half.at[src_idx], rows, sem).wait()
        pltpu.sync_copy(rows, agg_sh.at[dst_idx], add=True)

        do_deg = jnp.logical_xor(ci < DEG_SPLIT, cid == 1)

        @pl.when(do_deg)
        def _count():
            pltpu.sync_copy(ones_mat, deg_sh.at[dst_idx], add=True)
        return _
    lax.fori_loop(0, NFULL, chunk_body, None)

    teoff = ebase + NFULL * CHUNK
    pltpu.sync_copy(src_hbm.at[pl.ds(teoff, TAIL)], tsrc_idx)
    pltpu.sync_copy(dst_hbm.at[pl.ds(teoff, TAIL)], tdst_idx)
    pltpu.async_copy(my_half.at[tsrc_idx], trows, sem).wait()
    pltpu.sync_copy(trows, agg_sh.at[tdst_idx], add=True)

    @pl.when(cid == 1)
    def _count_tail():
        pltpu.sync_copy(ones_mat.at[pl.ds(0, TAIL)],
                        deg_sh.at[tdst_idx], add=True)

    plsc.subcore_barrier()

    wbase = sid * 624
    pltpu.sync_copy(agg_sh.at[pl.ds(wbase, 624)],
                    agg_hbm.at[cid, pl.ds(wbase, 624)])
    pltpu.sync_copy(deg_sh.at[pl.ds(wbase, 624)],
                    deg_hbm.at[cid, pl.ds(wbase, 624)])

    @pl.when(sid == NS - 1)
    def _write_tail():
        pltpu.sync_copy(agg_sh.at[pl.ds(9984, 16)],
                        agg_hbm.at[cid, pl.ds(9984, 16)])
        pltpu.sync_copy(deg_sh.at[pl.ds(9984, 16)],
                        deg_hbm.at[cid, pl.ds(9984, 16)])


_sc_aggregate = functools.partial(
    pl.kernel,
    out_type=(jax.ShapeDtypeStruct((NC, N, DH), jnp.float32),
              jax.ShapeDtypeStruct((NC, N, 16), jnp.float32)),
    mesh=plsc.VectorSubcoreMesh(core_axis_name="c", subcore_axis_name="s"),
    compiler_params=pltpu.CompilerParams(use_tc_tiling_on_sc=False),
    scratch_types=[
        pltpu.VMEM((CHUNK,), jnp.int32),
        pltpu.VMEM((CHUNK,), jnp.int32),
        pltpu.VMEM((CHUNK, DH), jnp.float32),
        pltpu.VMEM((TAIL,), jnp.int32),
        pltpu.VMEM((TAIL,), jnp.int32),
        pltpu.VMEM((TAIL, DH), jnp.float32),
        pltpu.VMEM((CHUNK, 16), jnp.float32),
        pltpu.VMEM((CHUNK, DH), jnp.float32),
        pltpu.VMEM((CHUNK, 16), jnp.float32),
        pltpu.VMEM_SHARED((N, DH), jnp.float32),
        pltpu.VMEM_SHARED((N, 16), jnp.float32),
        pltpu.SemaphoreType.DMA,
    ],
)(_sc_body)


def _tc_body(agg_ref, deg_ref, w_ref, b_ref, o_ref):
    a = jnp.concatenate([agg_ref[0], agg_ref[1]], axis=1)
    d = deg_ref[0] + deg_ref[1]
    inv = 1.0 / jnp.maximum(d[:, :1], 1.0)
    h = jnp.dot(a, w_ref[...], preferred_element_type=jnp.float32)
    o_ref[...] = h * inv + b_ref[...][None, :]


BLK = 1000


def _tc_finish(agg2, deg2, W, b):
    return pl.pallas_call(
        _tc_body,
        grid=(N // BLK,),
        in_specs=[
            pl.BlockSpec((NC, BLK, DH), lambda i: (0, i, 0)),
            pl.BlockSpec((NC, BLK, 16), lambda i: (0, i, 0)),
            pl.BlockSpec((D, D), lambda i: (0, 0)),
            pl.BlockSpec((D,), lambda i: (0,)),
        ],
        out_specs=pl.BlockSpec((BLK, D), lambda i: (i, 0)),
        out_shape=jax.ShapeDtypeStruct((N, D), jnp.float32),
    )(agg2, deg2, W, b)


def kernel(x, edge_index, W, b):
    src = edge_index[0]
    dst = edge_index[1]
    xs = x.reshape(N, NC, DH).transpose(1, 0, 2)
    agg2, deg2 = _sc_aggregate(xs, src, dst)
    return _tc_finish(agg2, deg2, W, b)


# speedup vs baseline: 12.0529x; 1.1832x over previous
import functools

import jax
import jax.numpy as jnp
from jax import lax
from jax.experimental import pallas as pl
from jax.experimental.pallas import tpu as pltpu
from jax.experimental.pallas import tpu_sc as plsc

N = 10000
E = 320000
D = 128
DH = D // 2

NC = 2
NS = 16
CHUNK = 128
CPT = 160
NCH = CPT * NS
DEG_SPLIT = CPT // 2
NBUF = 4
NA = N + 16
ROWS_PER_TILE = NA // NS


def _sc_body(xs_hbm, src_hbm, dst_hbm, agg_hbm, deg_hbm,
             src_all, dst_all, buf0, buf1, buf2, buf3,
             ones_mat, zdeg, agg_sh, deg_sh,
             sem0, sem1, sem2, sem3):
    cid = lax.axis_index("c")
    sid = lax.axis_index("s")
    bufs = (buf0, buf1, buf2, buf3)
    sems = (sem0, sem1, sem2, sem3)

    zero16 = jnp.zeros((16,), jnp.float32)
    ones16 = jnp.full((16,), 1.0, jnp.float32)

    def init_row(i, _):
        ones_mat[i, :] = ones16
        zdeg[i, :] = zero16
        for j in range(DH // 16):
            buf0[i, pl.ds(j * 16, 16)] = zero16
        return _
    lax.fori_loop(0, CHUNK, init_row, None)

    my_half = xs_hbm.at[cid]
    cbase = sid * CPT
    pltpu.sync_copy(src_hbm.at[pl.ds(cbase, CPT)], src_all)
    pltpu.sync_copy(dst_hbm.at[pl.ds(cbase, CPT)], dst_all)

    base_row = sid * ROWS_PER_TILE
    off = 0
    for sz in (128, 128, 128, 128, 114):
        pltpu.sync_copy(buf0.at[pl.ds(0, sz)],
                        agg_sh.at[pl.ds(base_row + off, sz)])
        pltpu.sync_copy(zdeg.at[pl.ds(0, sz)],
                        deg_sh.at[pl.ds(base_row + off, sz)])
        off += sz

    for b in range(NBUF):
        pltpu.async_copy(my_half.at[src_all.at[b]], bufs[b], sems[b])
    plsc.subcore_barrier()

    def step(ci, buf, sem, issue_next):
        pltpu.make_async_copy(my_half.at[src_all.at[ci]], buf, sem).wait()
        pltpu.sync_copy(buf, agg_sh.at[dst_all.at[ci]], add=True)

        do_deg = jnp.logical_xor(ci < DEG_SPLIT, cid == 1)

        @pl.when(do_deg)
        def _count():
            pltpu.sync_copy(ones_mat, deg_sh.at[dst_all.at[ci]], add=True)

        if issue_next:
            pltpu.async_copy(my_half.at[src_all.at[ci + NBUF]], buf, sem)

    def body(g, _):
        for b in range(NBUF):
            step(g * NBUF + b, bufs[b], sems[b], True)
        return _
    lax.fori_loop(0, CPT // NBUF - 1, body, None)

    for b in range(NBUF):
        step(CPT - NBUF + b, bufs[b], sems[b], False)

    plsc.subcore_barrier()

    wbase = sid * 624
    pltpu.sync_copy(agg_sh.at[pl.ds(wbase, 624)],
                    agg_hbm.at[cid, pl.ds(wbase, 624)])
    pltpu.sync_copy(deg_sh.at[pl.ds(wbase, 624)],
                    deg_hbm.at[cid, pl.ds(wbase, 624)])

    @pl.when(sid == NS - 1)
    def _write_tail():
        pltpu.sync_copy(agg_sh.at[pl.ds(9984, 16)],
                        agg_hbm.at[cid, pl.ds(9984, 16)])
        pltpu.sync_copy(deg_sh.at[pl.ds(9984, 16)],
                        deg_hbm.at[cid, pl.ds(9984, 16)])


_sc_aggregate = functools.partial(
    pl.kernel,
    out_type=(jax.ShapeDtypeStruct((NC, N, DH), jnp.float32),
              jax.ShapeDtypeStruct((NC, N, 16), jnp.float32)),
    mesh=plsc.VectorSubcoreMesh(core_axis_name="c", subcore_axis_name="s"),
    compiler_params=pltpu.CompilerParams(use_tc_tiling_on_sc=False),
    scratch_types=[
        pltpu.VMEM((CPT, CHUNK), jnp.int32),
        pltpu.VMEM((CPT, CHUNK), jnp.int32),
        pltpu.VMEM((CHUNK, DH), jnp.float32),
        pltpu.VMEM((CHUNK, DH), jnp.float32),
        pltpu.VMEM((CHUNK, DH), jnp.float32),
        pltpu.VMEM((CHUNK, DH), jnp.float32),
        pltpu.VMEM((CHUNK, 16), jnp.float32),
        pltpu.VMEM((CHUNK, 16), jnp.float32),
        pltpu.VMEM_SHARED((NA, DH), jnp.float32),
        pltpu.VMEM_SHARED((NA, 16), jnp.float32),
        pltpu.SemaphoreType.DMA,
        pltpu.SemaphoreType.DMA,
        pltpu.SemaphoreType.DMA,
        pltpu.SemaphoreType.DMA,
    ],
)(_sc_body)


def _tc_body(agg_ref, deg_ref, w_ref, b_ref, o_ref):
    a = jnp.concatenate([agg_ref[0], agg_ref[1]], axis=1)
    d = deg_ref[0] + deg_ref[1]
    inv = 1.0 / jnp.maximum(d[:, :1], 1.0)
    h = jnp.dot(a, w_ref[...], preferred_element_type=jnp.float32)
    o_ref[...] = h * inv + b_ref[...][None, :]


BLK = 1000


def _tc_finish(agg2, deg2, W, b):
    return pl.pallas_call(
        _tc_body,
        grid=(N // BLK,),
        in_specs=[
            pl.BlockSpec((NC, BLK, DH), lambda i: (0, i, 0)),
            pl.BlockSpec((NC, BLK, 16), lambda i: (0, i, 0)),
            pl.BlockSpec((D, D), lambda i: (0, 0)),
            pl.BlockSpec((D,), lambda i: (0,)),
        ],
        out_specs=pl.BlockSpec((BLK, D), lambda i: (i, 0)),
        out_shape=jax.ShapeDtypeStruct((N, D), jnp.float32),
    )(agg2, deg2, W, b)


def kernel(x, edge_index, W, b):
    src = edge_index[0]
    dst = edge_index[1]
    pad = NCH * CHUNK - E
    src_p = jnp.concatenate([src, jnp.zeros((pad,), jnp.int32)])
    dst_p = jnp.concatenate(
        [dst, N + (jnp.arange(pad, dtype=jnp.int32) % 16)])
    src2 = src_p.reshape(NCH, CHUNK)
    dst2 = dst_p.reshape(NCH, CHUNK)
    xs = x.reshape(N, NC, DH).transpose(1, 0, 2)
    agg2, deg2 = _sc_aggregate(xs, src2, dst2)
    return _tc_finish(agg2, deg2, W, b)
